# baseline (speedup 1.0000x reference)
import jax
import jax.numpy as jnp
from jax import lax
from jax.experimental import pallas as pl
from jax.experimental.pallas import tpu as pltpu
from jax.experimental.pallas import tpu_sc as plsc

HID = 64
NC, NS = 2, 16
NW = NC * NS
S = 50
CHUNK = 128
NTC = 16384 // CHUNK
TCW = NTC // NW
NG = S * TCW
PITCH = 2 * CHUNK + 1


def _gather_body(seq_hbm, table_hbm, out_hbm, idx_v, rows_v, t_v, gsem, osem):
    wid = lax.axis_index("s") * NC + lax.axis_index("c")
    pltpu.sync_copy(seq_hbm.at[:, pl.ds(wid * TCW, TCW)], idx_v)

    lanes = jax.lax.broadcasted_iota(jnp.int32, (16,), 0)
    cb_lanes = [lanes + cb for cb in range(0, HID, 16)]

    def g_copy(b, g):
        s, tcl = g // TCW, g % TCW
        return pltpu.make_async_copy(
            table_hbm.at[idx_v.at[s, tcl]],
            rows_v.at[pl.ds(b * CHUNK, CHUNK)], gsem.at[b])

    def o_copies(b, g):
        s, tcl = g // TCW, g % TCW
        return [pltpu.make_async_copy(
            t_v.at[pl.ds(tr * 8, 8), pl.ds(b * CHUNK, CHUNK)],
            out_hbm.at[s, tr, wid * TCW + tcl], osem.at[b])
            for tr in range(HID // 8)]

    g_copy(0, 0).start()

    def body(g, carry):
        b = g % 2
        nb = 1 - b
        boff = b * CHUNK

        @pl.when(g + 1 < NG)
        def _fire_next():
            g_copy(nb, g + 1).start()

        g_copy(b, g).wait()

        @pl.when(g >= 2)
        def _drain_old():
            for c in o_copies(b, g - 2):
                c.wait()

        for r0 in range(0, CHUNK, 4):
            ps = [jnp.full((16,), boff + r0 + i, jnp.int32) for i in range(4)]
            vs = [rows_v[boff + r0 + i, pl.ds(cb, 16)]
                  for i in range(4) for cb in range(0, HID, 16)]
            for i in range(4):
                for j in range(4):
                    plsc.store_scatter(t_v, [cb_lanes[j], ps[i]], vs[i * 4 + j])

        for c in o_copies(b, g):
            c.start()
        return carry

    lax.fori_loop(0, NG, body, 0)
    for c in o_copies((NG - 2) % 2, NG - 2):
        c.wait()
    for c in o_copies((NG - 1) % 2, NG - 1):
        c.wait()


def kernel(seq, table):
    seq3 = seq.T.reshape(S, NTC, CHUNK)
    out5 = pl.kernel(
        _gather_body,
        out_type=jax.ShapeDtypeStruct((S, HID // 8, NTC, 8, CHUNK), jnp.float32),
        mesh=plsc.VectorSubcoreMesh(core_axis_name="c", subcore_axis_name="s"),
        scratch_types=[
            pltpu.VMEM((S, TCW, CHUNK), jnp.int32),
            pltpu.VMEM((2 * CHUNK, HID), jnp.float32),
            pltpu.VMEM((HID, PITCH), jnp.float32),
            pltpu.SemaphoreType.DMA((2,)),
            pltpu.SemaphoreType.DMA((2,)),
        ],
        compiler_params=pltpu.CompilerParams(
            use_tc_tiling_on_sc=False, needs_layout_passes=False),
    )(seq3, table)
    return out5.transpose(2, 4, 0, 1, 3).reshape(16384, S, HID)


# speedup vs baseline: 1.0030x; 1.0030x over previous
import jax
import jax.numpy as jnp
from jax import lax
from jax.experimental import pallas as pl
from jax.experimental.pallas import tpu as pltpu
from jax.experimental.pallas import tpu_sc as plsc

HID = 64
NC, NS = 2, 16
NW = NC * NS
S = 50
CHUNK = 128
NTC = 16384 // CHUNK
TCW = NTC // NW
NG = S * TCW
PITCH = 2 * CHUNK + 1


def _gather_body(seq_hbm, table_hbm, out_hbm, idx_v, rows_v, t_v, gsem, osem):
    wid = lax.axis_index("s") * NC + lax.axis_index("c")
    pltpu.sync_copy(seq_hbm.at[:, pl.ds(wid * TCW, TCW)], idx_v)

    lanes = jax.lax.broadcasted_iota(jnp.int32, (16,), 0)
    cb_lanes = [lanes + cb for cb in range(0, HID, 16)]

    def g_copy(b, g):
        s, tcl = g // TCW, g % TCW
        return pltpu.make_async_copy(
            table_hbm.at[idx_v.at[s, tcl]],
            rows_v.at[pl.ds(b * CHUNK, CHUNK)], gsem.at[b])

    def o_copies(b, g):
        s, tcl = g // TCW, g % TCW
        return [pltpu.make_async_copy(
            t_v.at[pl.ds(tr * 8, 8), pl.ds(b * CHUNK, CHUNK)],
            out_hbm.at[s, tr, wid * TCW + tcl], osem.at[b])
            for tr in range(HID // 8)]

    g_copy(0, 0).start()

    def body(g, carry):
        b = g % 2
        nb = 1 - b
        boff = b * CHUNK

        @pl.when(g + 1 < NG)
        def _fire_next():
            g_copy(nb, g + 1).start()

        g_copy(b, g).wait()

        @pl.when(g >= 2)
        def _drain_old():
            for c in o_copies(b, g - 2):
                c.wait()

        for r0 in range(0, CHUNK, 2):
            p0 = lanes * 0 + (boff + r0)
            p1 = lanes * 0 + (boff + r0 + 1)
            vs = [rows_v[boff + r0, pl.ds(cb, 16)] for cb in range(0, HID, 16)]
            vs += [rows_v[boff + r0 + 1, pl.ds(cb, 16)] for cb in range(0, HID, 16)]
            for j in range(4):
                plsc.store_scatter(t_v, [cb_lanes[j], p0], vs[j])
            for j in range(4):
                plsc.store_scatter(t_v, [cb_lanes[j], p1], vs[4 + j])

        for c in o_copies(b, g):
            c.start()
        return carry

    lax.fori_loop(0, NG, body, 0)
    for c in o_copies((NG - 2) % 2, NG - 2):
        c.wait()
    for c in o_copies((NG - 1) % 2, NG - 1):
        c.wait()


def kernel(seq, table):
    seq3 = seq.T.reshape(S, NTC, CHUNK)
    out5 = pl.kernel(
        _gather_body,
        out_type=jax.ShapeDtypeStruct((S, HID // 8, NTC, 8, CHUNK), jnp.float32),
        mesh=plsc.VectorSubcoreMesh(core_axis_name="c", subcore_axis_name="s"),
        scratch_types=[
            pltpu.VMEM((S, TCW, CHUNK), jnp.int32),
            pltpu.VMEM((2 * CHUNK, HID), jnp.float32),
            pltpu.VMEM((HID, PITCH), jnp.float32),
            pltpu.SemaphoreType.DMA((2,)),
            pltpu.SemaphoreType.DMA((2,)),
        ],
        compiler_params=pltpu.CompilerParams(
            use_tc_tiling_on_sc=False, needs_layout_passes=False),
    )(seq3, table)
    return out5.transpose(2, 4, 0, 1, 3).reshape(16384, S, HID)
